# baseline (speedup 1.0000x reference)
import jax
import jax.numpy as jnp
from jax import lax
from jax.experimental import pallas as pl
from jax.experimental.pallas import tpu as pltpu
from jax.experimental.pallas import tpu_sc as plsc

B = 4096
F = 26
V = 100000
K = 64
H1 = 512
H2 = 256
D = F * K
BF = B * F
NC, NS = 2, 16
NW = NC * NS
ROWS_PER_W = BF // NW
CHUNK = 128
NCH = ROWS_PER_W // CHUNK


def _sc_gather_body(idx_hbm, table_hbm, out_hbm, idx_v, rows_v, sem):
    wid = lax.axis_index("s") * NC + lax.axis_index("c")
    pltpu.sync_copy(idx_hbm.at[wid], idx_v)
    base = wid * ROWS_PER_W

    def body(j, carry):
        pltpu.async_copy(table_hbm.at[idx_v.at[j]], rows_v, sem).wait()
        pltpu.sync_copy(rows_v, out_hbm.at[pl.ds(base + j * CHUNK, CHUNK)])
        return carry

    lax.fori_loop(0, NCH, body, 0)


_SC_GATHER_CACHE = {}


def _sc_gather():
    if "k" not in _SC_GATHER_CACHE:
        _SC_GATHER_CACHE["k"] = pl.kernel(
            _sc_gather_body,
            out_type=jax.ShapeDtypeStruct((BF, K), jnp.float32),
            mesh=plsc.VectorSubcoreMesh(
                core_axis_name="c", subcore_axis_name="s",
                num_cores=NC, num_subcores=NS,
            ),
            scratch_types=[
                pltpu.VMEM((NCH, CHUNK), jnp.int32),
                pltpu.VMEM((CHUNK, K), jnp.float32),
                pltpu.SemaphoreType.DMA,
            ],
            compiler_params=pltpu.CompilerParams(use_tc_tiling_on_sc=False),
        )
    return _SC_GATHER_CACHE["k"]

BB = 512


def _dense_body(x_ref, b0_ref, w1_ref, b1_ref, w2_ref, b2_ref, w3_ref, b3_ref,
                o_ref):
    l = jnp.tanh(x_ref[...] + b0_ref[...])
    h1 = jnp.dot(l, w1_ref[...], preferred_element_type=jnp.float32)
    h1 = jnp.maximum(h1 + b1_ref[...], 0.0)
    h2 = jnp.dot(h1, w2_ref[...], preferred_element_type=jnp.float32)
    h2 = jnp.maximum(h2 + b2_ref[...], 0.0)
    y = jnp.sum(h2 * w3_ref[...], axis=1) + b3_ref[0]
    o_ref[...] = jax.nn.sigmoid(y)


_dense = pl.pallas_call(
    _dense_body,
    grid=(B // BB,),
    in_specs=[
        pl.BlockSpec((BB, D), lambda i: (i, 0)),
        pl.BlockSpec((1, D), lambda i: (0, 0)),
        pl.BlockSpec((D, H1), lambda i: (0, 0)),
        pl.BlockSpec((1, H1), lambda i: (0, 0)),
        pl.BlockSpec((H1, H2), lambda i: (0, 0)),
        pl.BlockSpec((1, H2), lambda i: (0, 0)),
        pl.BlockSpec((1, H2), lambda i: (0, 0)),
        pl.BlockSpec(memory_space=pltpu.SMEM),
    ],
    out_specs=pl.BlockSpec((BB,), lambda i: (i,)),
    out_shape=jax.ShapeDtypeStruct((B,), jnp.float32),
    compiler_params=pltpu.CompilerParams(
        dimension_semantics=("arbitrary",),
    ),
)


def kernel(indices, W0, b0, w1, k1, b1, w2, b2, w3, b3):
    flat_idx = (indices.astype(jnp.int32)
                + (jnp.arange(F, dtype=jnp.int32) * V)[None, :])
    emb = _sc_gather()(flat_idx.reshape(NW, NCH, CHUNK), W0.reshape(F * V, K))
    w1p = w1 + jnp.repeat(k1, K, axis=0)
    return _dense(emb.reshape(B, D), b0.reshape(1, D), w1p, b1.reshape(1, H1),
                  w2, b2.reshape(1, H2), w3.reshape(1, H2), b3)
